# baseline (speedup 1.0000x reference)
import functools

import jax
import jax.numpy as jnp
from jax import lax
from jax.experimental import pallas as pl
from jax.experimental.pallas import tpu as pltpu
from jax.experimental.pallas import tpu_sc as plsc

NC = 2
NS = 16
NW = NC * NS
LANES = 16


def _make_sc_gather(B, D, NEG):
    per_w = B // NW
    CH = per_w // 2
    ngrp = CH // LANES

    mesh = plsc.VectorSubcoreMesh(core_axis_name="c", subcore_axis_name="s")

    @functools.partial(
        pl.kernel,
        out_type=[
            jax.ShapeDtypeStruct((B, D), jnp.float32),
            jax.ShapeDtypeStruct((B, D), jnp.float32),
            jax.ShapeDtypeStruct((NEG * B, D), jnp.float32),
        ],
        mesh=mesh,
        compiler_params=pltpu.CompilerParams(use_tc_tiling_on_sc=False),
        scratch_types=[
            pltpu.VMEM((CH,), jnp.int32),
            pltpu.VMEM((CH, 64), jnp.float32),
            pltpu.VMEM((CH, 64), jnp.float32),
            pltpu.SemaphoreType.DMA,
            pltpu.SemaphoreType.DMA,
            pltpu.SemaphoreType.DMA,
        ],
    )
    def sc_fn(pos_u_h, pos_v_h, negt_h, uw_h, vw_h, eu_h, ev_h, en_h,
              idx_v, rows0, rows1, gsem, wsem0, wsem1):
        wid = lax.axis_index("s") * NC + lax.axis_index("c")
        base = pl.multiple_of(wid * per_w, per_w)
        bufs = (rows0, rows1)
        wsems = (wsem0, wsem1)

        tasks = []
        for idx_h, table, out, off0 in (
                [(pos_u_h, uw_h, eu_h, 0), (pos_v_h, vw_h, ev_h, 0)]
                + [(negt_h, vw_h, en_h, j * B) for j in range(NEG)]):
            for c in range(per_w // CH):
                off = pl.multiple_of(off0 + base + c * CH, CH)
                tasks.append((idx_h, off, table, out, off))

        for t, (idx_h, idx_off, table, out, out_off) in enumerate(tasks):
            buf = bufs[t % 2]
            wsem = wsems[t % 2]
            if t >= 2:
                pltpu.make_async_copy(buf, out.at[pl.ds(out_off, CH)],
                                      wsem).wait()
            pltpu.sync_copy(idx_h.at[pl.ds(idx_off, CH)], idx_v)

            def group(g, c, buf=buf, table=table):
                o = pl.multiple_of(g * LANES, LANES)
                vec = idx_v[pl.ds(o, LANES)]
                for i in range(LANES):
                    r = vec[i]
                    pltpu.async_copy(table.at[pl.ds(r, 1)],
                                     buf.at[pl.ds(o + i, 1)], gsem)
                return c
            lax.fori_loop(0, ngrp, group, 0)
            pltpu.make_async_copy(table.at[pl.ds(0, CH)], buf, gsem).wait()
            pltpu.async_copy(buf, out.at[pl.ds(out_off, CH)], wsem)

        for t in (len(tasks) - 2, len(tasks) - 1):
            _, _, _, out, out_off = tasks[t]
            pltpu.make_async_copy(bufs[t % 2], out.at[pl.ds(out_off, CH)],
                                  wsems[t % 2]).wait()

    return sc_fn


def _softplus(x):
    return jnp.maximum(x, 0.0) + jnp.log1p(jnp.exp(-jnp.abs(x)))


def _make_tc_loss(B, D, NEG, BT):
    nsteps = B // BT

    def body(eu_ref, ev_ref, en_ref, out_ref):
        i = pl.program_id(0)
        eu = eu_ref[...]
        ev = ev_ref[...]
        s = jnp.clip(jnp.sum(eu * ev, axis=1), -10.0, 10.0)
        acc = jnp.sum(_softplus(-s))
        for j in range(NEG):
            nj = en_ref[j]
            t = jnp.clip(jnp.sum(nj * eu, axis=1), -10.0, 10.0)
            acc = acc + jnp.sum(_softplus(t))

        @pl.when(i == 0)
        def _():
            out_ref[0, 0] = 0.0
        out_ref[0, 0] += acc / B

    return pl.pallas_call(
        body,
        grid=(nsteps,),
        in_specs=[
            pl.BlockSpec((BT, D), lambda i: (i, 0)),
            pl.BlockSpec((BT, D), lambda i: (i, 0)),
            pl.BlockSpec((NEG, BT, D), lambda i: (0, i, 0)),
        ],
        out_specs=pl.BlockSpec((1, 1), lambda i: (0, 0),
                               memory_space=pltpu.SMEM),
        out_shape=jax.ShapeDtypeStruct((1, 1), jnp.float32),
    )


def kernel(pos_u, pos_v, neg_v, u_weight, v_weight):
    B = pos_u.shape[0]
    NEG = neg_v.shape[1]
    D = u_weight.shape[1]

    negt = neg_v.T.reshape(-1)
    sc_fn = _make_sc_gather(B, D, NEG)
    eu, ev, en = sc_fn(pos_u, pos_v, negt, u_weight, v_weight)

    en3 = en.reshape(NEG, B, D)
    out = _make_tc_loss(B, D, NEG, BT=2048)(eu, ev, en3)
    return out[0, 0]


# speedup vs baseline: 1.5575x; 1.5575x over previous
import functools

import jax
import jax.numpy as jnp
from jax import lax
from jax.experimental import pallas as pl
from jax.experimental.pallas import tpu as pltpu
from jax.experimental.pallas import tpu_sc as plsc

NC = 2
NS = 16
NW = NC * NS
LANES = 16


def _make_sc_gather(B, D, NEG):
    per_w = B // NW
    CH = per_w // 2
    ngrp = CH // LANES

    mesh = plsc.VectorSubcoreMesh(core_axis_name="c", subcore_axis_name="s")

    @functools.partial(
        pl.kernel,
        out_type=[
            jax.ShapeDtypeStruct((B, D), jnp.float32),
            jax.ShapeDtypeStruct((B, D), jnp.float32),
            jax.ShapeDtypeStruct((NEG * B, D), jnp.float32),
        ],
        mesh=mesh,
        scratch_types=[
            pltpu.VMEM((CH,), jnp.int32),
            pltpu.VMEM((CH, 64), jnp.float32),
            pltpu.VMEM((CH, 64), jnp.float32),
            pltpu.SemaphoreType.DMA,
            pltpu.SemaphoreType.DMA,
            pltpu.SemaphoreType.DMA,
        ],
    )
    def sc_fn(pos_u_h, pos_v_h, negt_h, uw_h, vw_h, eu_h, ev_h, en_h,
              idx_v, rows0, rows1, gsem, wsem0, wsem1):
        wid = lax.axis_index("s") * NC + lax.axis_index("c")
        base = pl.multiple_of(wid * per_w, per_w)
        bufs = (rows0, rows1)
        wsems = (wsem0, wsem1)

        tasks = []
        for idx_h, table, out, off0 in (
                [(pos_u_h, uw_h, eu_h, 0), (pos_v_h, vw_h, ev_h, 0)]
                + [(negt_h, vw_h, en_h, j * B) for j in range(NEG)]):
            for c in range(per_w // CH):
                off = pl.multiple_of(off0 + base + c * CH, CH)
                tasks.append((idx_h, off, table, out, off))

        for t, (idx_h, idx_off, table, out, out_off) in enumerate(tasks):
            buf = bufs[t % 2]
            wsem = wsems[t % 2]
            if t >= 2:
                pltpu.make_async_copy(buf, out.at[pl.ds(out_off, CH)],
                                      wsem).wait()
            pltpu.sync_copy(idx_h.at[pl.ds(idx_off, CH)], idx_v)

            def group(g, c, buf=buf, table=table):
                o = pl.multiple_of(g * LANES, LANES)
                vec = idx_v[pl.ds(o, LANES)]
                for i in range(LANES):
                    r = vec[i]
                    pltpu.async_copy(table.at[pl.ds(r, 1)],
                                     buf.at[pl.ds(o + i, 1)], gsem)
                return c
            lax.fori_loop(0, ngrp, group, 0)
            pltpu.make_async_copy(table.at[pl.ds(0, CH)], buf, gsem).wait()
            pltpu.async_copy(buf, out.at[pl.ds(out_off, CH)], wsem)

        for t in (len(tasks) - 2, len(tasks) - 1):
            _, _, _, out, out_off = tasks[t]
            pltpu.make_async_copy(bufs[t % 2], out.at[pl.ds(out_off, CH)],
                                  wsems[t % 2]).wait()

    return sc_fn


def _softplus(x):
    return jnp.maximum(x, 0.0) + jnp.log1p(jnp.exp(-jnp.abs(x)))


def _make_tc_loss(B, D, NEG, BT):
    nsteps = B // BT

    def body(eu_ref, ev_ref, en_ref, out_ref):
        i = pl.program_id(0)
        eu = eu_ref[...]
        ev = ev_ref[...]
        s = jnp.clip(jnp.sum(eu * ev, axis=1), -10.0, 10.0)
        acc = jnp.sum(_softplus(-s))
        for j in range(NEG):
            nj = en_ref[j]
            t = jnp.clip(jnp.sum(nj * eu, axis=1), -10.0, 10.0)
            acc = acc + jnp.sum(_softplus(t))

        @pl.when(i == 0)
        def _():
            out_ref[0, 0] = 0.0
        out_ref[0, 0] += acc / B

    return pl.pallas_call(
        body,
        grid=(nsteps,),
        in_specs=[
            pl.BlockSpec((BT, D), lambda i: (i, 0)),
            pl.BlockSpec((BT, D), lambda i: (i, 0)),
            pl.BlockSpec((NEG, BT, D), lambda i: (0, i, 0)),
        ],
        out_specs=pl.BlockSpec((1, 1), lambda i: (0, 0),
                               memory_space=pltpu.SMEM),
        out_shape=jax.ShapeDtypeStruct((1, 1), jnp.float32),
    )


def kernel(pos_u, pos_v, neg_v, u_weight, v_weight):
    B = pos_u.shape[0]
    NEG = neg_v.shape[1]
    D = u_weight.shape[1]

    negt = neg_v.T.reshape(-1)
    sc_fn = _make_sc_gather(B, D, NEG)
    eu, ev, en = sc_fn(pos_u, pos_v, negt, u_weight, v_weight)

    en3 = en.reshape(NEG, B, D)
    out = _make_tc_loss(B, D, NEG, BT=2048)(eu, ev, en3)
    return out[0, 0]
